# baseline (speedup 1.0000x reference)
import functools

import jax
import jax.numpy as jnp
import numpy as np
from jax import lax
from jax.experimental import pallas as pl
from jax.experimental.pallas import tpu as pltpu
from jax.experimental.pallas import tpu_sc as plsc

_OFFSET = 2
_V = 2048 + _OFFSET
_D = 1024
_P = 8
_B = 2
_S = 2048

_NC = 2
_NS = 16
_NW = _NC * _NS
_SB = _S // _NW
_K = 16
_NCHUNK = _SB // _K
_NSTEP = _NCHUNK * _P

_DW = _D // 4


def _packed_sign_table() -> np.ndarray:
    def draw():
        key = jax.random.key(42)
        return np.asarray(jax.random.randint(key, (_P, _V, _D), 0, 2))

    try:
        with jax.default_device(jax.devices("cpu")[0]):
            mu01 = draw()
    except Exception:
        try:
            mu01 = draw()
        except Exception:
            mu01 = np.zeros((_P, _V, _D), np.int64)
    sign = ((1 - mu01) * 0x80).astype(np.uint32).reshape(_P * _V, 16, 4, 16)
    packed = (sign[:, :, 0, :] | (sign[:, :, 1, :] << 8)
              | (sign[:, :, 2, :] << 16) | (sign[:, :, 3, :] << 24))
    return np.ascontiguousarray(
        packed.reshape(_P * _V, _DW).view(np.int32))


def _bit_table(byte_table: np.ndarray) -> np.ndarray:
    bits = ((byte_table.view(np.uint8).reshape(_P * _V, _DW, 4) >> 7) & 1
            ).astype(np.uint32)
    el = np.zeros((_P * _V, _D), np.uint32)
    g = np.arange(_DW) // 16
    j = np.arange(_DW) % 16
    for r in range(4):
        el[:, g * 64 + r * 16 + j] = bits[:, :, r]
    el = el.reshape(_P * _V, _D // 32, 32)
    packed = (el << np.arange(32, dtype=np.uint32)).sum(
        axis=2, dtype=np.uint32)
    return np.ascontiguousarray(packed.view(np.int32))


_ESIGN_BITS = _bit_table(_packed_sign_table())


def _expand_sign_table(bits):
    bits = lax.optimization_barrier(jnp.asarray(bits))
    b0 = bits[:, 0::2][:, :, None]
    b1 = bits[:, 1::2][:, :, None]
    j = jnp.arange(16, dtype=jnp.int32)[None, None, :]
    word = ((((b0 >> j) & 1) << 7)
            | (((b0 >> (16 + j)) & 1) << 15)
            | (((b1 >> j) & 1) << 23)
            | (((b1 >> (16 + j)) & 1) << 31))
    return word.reshape(_P * _V, _DW)


def _sc_body(mask_hbm, w_hbm, emu_hbm, out_hbm,
             mask_v, pos_v, eidx, wbuf, ebuf, obuf, sem_w, sem_e, sem_o):
    cid = lax.axis_index("c")
    sid = lax.axis_index("s")
    wid = sid * _NC + cid
    s0 = wid * _SB
    c0 = wid * _NCHUNK

    pltpu.sync_copy(mask_hbm, mask_v)

    def scan_body(c, carry):
        cy0, cy1 = carry
        m0 = mask_v[0, pl.ds(c * 16, 16)]
        m1 = mask_v[1, pl.ds(c * 16, 16)]
        cs0 = plsc.cumsum(m0) + cy0
        cs1 = plsc.cumsum(m1) + cy1

        @pl.when(jnp.logical_and(c >= c0, c < c0 + _NCHUNK))
        def _():
            pos_v[0, pl.ds((c - c0) * 16, 16)] = cs0 * m0 + (_OFFSET - 1)
            pos_v[1, pl.ds((c - c0) * 16, 16)] = cs1 * m1 + (_OFFSET - 1)

        return cs0[15], cs1[15]

    lax.fori_loop(0, _S // 16, scan_body, (jnp.int32(0), jnp.int32(0)))

    eq = jnp.bool_(True)
    for cc in range(_NCHUNK):
        sl = pl.ds(cc * 16, 16)
        eq = jnp.logical_and(eq, jnp.all(pos_v[0, sl] == pos_v[1, sl]))

    def fire_w(c, kc, bb):
        pltpu.async_copy(w_hbm.at[pos_v.at[bb, pl.ds(c * _K, _K)]],
                         wbuf[kc], sem_w[kc])

    def wait_w(c, kc, bb):
        pltpu.make_async_copy(w_hbm.at[pos_v.at[bb, pl.ds(c * _K, _K)]],
                              wbuf[kc], sem_w[kc]).wait()

    def fire_e(t, ke, bb):
        c = t // _P
        p = t % _P
        eidx[ke][...] = pos_v[bb, pl.ds(c * _K, _K)] + p * _V
        pltpu.async_copy(emu_hbm.at[eidx[ke]], ebuf[ke], sem_e[ke])

    def wait_e(ke):
        pltpu.make_async_copy(emu_hbm.at[eidx[ke]], ebuf[ke],
                              sem_e[ke]).wait()

    def row_of(t, bb):
        c = t // _P
        p = t % _P
        return (p * _B + bb) * _S + s0 + c * _K

    def compute(kc, kp, ko):
        def row_body(r, _3):
            for g in range(_D // 64):
                wg = ebuf[kp][r, pl.ds(g * 16, 16)]
                for rr in range(4):
                    shifted = jnp.left_shift(wg, 24 - 8 * rr)
                    pert = jnp.where(shifted < 0,
                                     jnp.float32(-0.01),
                                     jnp.float32(0.01))
                    sl = pl.ds(g * 64 + rr * 16, 16)
                    obuf[ko][r, sl] = wbuf[kc][r, sl] + pert
            return 0

        lax.fori_loop(0, _K, row_body, 0)

    @pl.when(eq)
    def _fast():
        fire_w(0, 0, 0)
        fire_e(0, 0, 0)

        def chunk_pair(ci, _):
            for kc in (0, 1):
                c = 2 * ci + kc

                @pl.when(c + 1 < _NCHUNK)
                def _():
                    fire_w(c + 1, 1 - kc, 0)

                wait_w(c, kc, 0)

                def p_pair(pj, _2):
                    for kp in (0, 1):
                        p = 2 * pj + kp
                        t = c * _P + p

                        @pl.when(t + 1 < _NSTEP)
                        def _():
                            fire_e(t + 1, 1 - kp, 0)

                        wait_e(kp)

                        @pl.when(t >= 2)
                        def _():
                            for bb in (0, 1):
                                pltpu.make_async_copy(
                                    obuf[kp],
                                    out_hbm.at[pl.ds(row_of(t - 2, bb), _K)],
                                    sem_o[kp]).wait()

                        compute(kc, kp, kp)
                        for bb in (0, 1):
                            pltpu.async_copy(
                                obuf[kp],
                                out_hbm.at[pl.ds(row_of(t, bb), _K)],
                                sem_o[kp])
                    return 0

                lax.fori_loop(0, _P // 2, p_pair, 0)
            return 0

        lax.fori_loop(0, _NCHUNK // 2, chunk_pair, 0)
        for t in (_NSTEP - 2, _NSTEP - 1):
            for bb in (0, 1):
                pltpu.make_async_copy(
                    obuf[t % 2], out_hbm.at[pl.ds(row_of(t, bb), _K)],
                    sem_o[t % 2]).wait()

    @pl.when(jnp.logical_not(eq))
    def _slow():
        def step_body(u, _):
            bb = u // _NSTEP
            t = u % _NSTEP
            c = t // _P
            p = t % _P

            @pl.when(p == 0)
            def _():
                fire_w(c, 0, bb)
                wait_w(c, 0, bb)

            fire_e(t, 0, bb)
            wait_e(0)
            compute(0, 0, 0)
            pltpu.sync_copy(obuf[0], out_hbm.at[pl.ds(row_of(t, bb), _K)])
            return 0

        lax.fori_loop(0, _B * _NSTEP, step_body, 0)


@functools.cache
def _sc_call():
    return pl.kernel(
        _sc_body,
        out_type=jax.ShapeDtypeStruct((_P * _B * _S, _D), jnp.float32),
        mesh=plsc.VectorSubcoreMesh(core_axis_name="c", subcore_axis_name="s",
                                    num_cores=_NC, num_subcores=_NS),
        compiler_params=pltpu.CompilerParams(needs_layout_passes=False),
        scratch_types=[
            pltpu.VMEM((_B, _S), jnp.int32),
            pltpu.VMEM((_B, _SB), jnp.int32),
            [pltpu.VMEM((_K,), jnp.int32)] * 2,
            [pltpu.VMEM((_K, _D), jnp.float32)] * 2,
            [pltpu.VMEM((_K, _DW), jnp.int32)] * 2,
            [pltpu.VMEM((_K, _D), jnp.float32)] * 2,
            [pltpu.SemaphoreType.DMA] * 2,
            [pltpu.SemaphoreType.DMA] * 2,
            [pltpu.SemaphoreType.DMA] * 2,
        ],
    )


def kernel(attention_mask, weight, past_key_values_length):
    del past_key_values_length
    mask = attention_mask.astype(jnp.int32)
    esign = _expand_sign_table(_ESIGN_BITS)
    out = _sc_call()(mask, weight.astype(jnp.float32), esign)
    return out.reshape(_P, _B, _S, _D)


# speedup vs baseline: 1.6184x; 1.6184x over previous
import functools

import jax
import jax.numpy as jnp
import numpy as np
from jax import lax
from jax.experimental import pallas as pl
from jax.experimental.pallas import tpu as pltpu
from jax.experimental.pallas import tpu_sc as plsc

_OFFSET = 2
_V = 2048 + _OFFSET
_D = 1024
_P = 8
_B = 2
_S = 2048

_NC = 2
_NS = 16
_NW = _NC * _NS
_SB = _S // _NW
_K = 16
_NCHUNK = _SB // _K
_NSTEP = _NCHUNK * _P

_DW = _D // 4


def _packed_sign_table() -> np.ndarray:
    def draw():
        key = jax.random.key(42)
        return np.asarray(jax.random.randint(key, (_P, _V, _D), 0, 2))

    try:
        with jax.default_device(jax.devices("cpu")[0]):
            mu01 = draw()
    except Exception:
        try:
            mu01 = draw()
        except Exception:
            mu01 = np.zeros((_P, _V, _D), np.int64)
    sign = ((1 - mu01) * 0x80).astype(np.uint32).reshape(_P * _V, 16, 4, 16)
    packed = (sign[:, :, 0, :] | (sign[:, :, 1, :] << 8)
              | (sign[:, :, 2, :] << 16) | (sign[:, :, 3, :] << 24))
    return np.ascontiguousarray(
        packed.reshape(_P * _V, _DW).view(np.int32))


_ESIGN = _packed_sign_table()
_ESIGN_DEV: dict = {}


def _esign_on_device():
    if "x" not in _ESIGN_DEV:
        _ESIGN_DEV["x"] = jax.device_put(_ESIGN)
    return _ESIGN_DEV["x"]


def _sc_body(mask_hbm, w_hbm, emu_hbm, out_hbm,
             mask_v, pos_v, eidx, wbuf, ebuf, obuf, sem_w, sem_e, sem_o):
    cid = lax.axis_index("c")
    sid = lax.axis_index("s")
    wid = sid * _NC + cid
    s0 = wid * _SB
    c0 = wid * _NCHUNK

    pltpu.sync_copy(mask_hbm, mask_v)

    def scan_body(c, carry):
        cy0, cy1 = carry
        m0 = mask_v[0, pl.ds(c * 16, 16)]
        m1 = mask_v[1, pl.ds(c * 16, 16)]
        cs0 = plsc.cumsum(m0) + cy0
        cs1 = plsc.cumsum(m1) + cy1

        @pl.when(jnp.logical_and(c >= c0, c < c0 + _NCHUNK))
        def _():
            pos_v[0, pl.ds((c - c0) * 16, 16)] = cs0 * m0 + (_OFFSET - 1)
            pos_v[1, pl.ds((c - c0) * 16, 16)] = cs1 * m1 + (_OFFSET - 1)

        return cs0[15], cs1[15]

    lax.fori_loop(0, _S // 16, scan_body, (jnp.int32(0), jnp.int32(0)))

    eq = jnp.bool_(True)
    for cc in range(_NCHUNK):
        sl = pl.ds(cc * 16, 16)
        eq = jnp.logical_and(eq, jnp.all(pos_v[0, sl] == pos_v[1, sl]))

    def fire_w(c, kc, bb):
        pltpu.async_copy(w_hbm.at[pos_v.at[bb, pl.ds(c * _K, _K)]],
                         wbuf[kc], sem_w[kc])

    def wait_w(c, kc, bb):
        pltpu.make_async_copy(w_hbm.at[pos_v.at[bb, pl.ds(c * _K, _K)]],
                              wbuf[kc], sem_w[kc]).wait()

    def fire_e(t, ke, bb):
        c = t // _P
        p = t % _P
        eidx[ke][...] = pos_v[bb, pl.ds(c * _K, _K)] + p * _V
        pltpu.async_copy(emu_hbm.at[eidx[ke]], ebuf[ke], sem_e[ke])

    def wait_e(ke):
        pltpu.make_async_copy(emu_hbm.at[eidx[ke]], ebuf[ke],
                              sem_e[ke]).wait()

    def row_of(t, bb):
        c = t // _P
        p = t % _P
        return (p * _B + bb) * _S + s0 + c * _K

    def compute(kc, kp, ko):
        def row_body(r, _3):
            for g in range(_D // 64):
                wg = ebuf[kp][r, pl.ds(g * 16, 16)]
                for rr in range(4):
                    shifted = jnp.left_shift(wg, 24 - 8 * rr)
                    pert = jnp.where(shifted < 0,
                                     jnp.float32(-0.01),
                                     jnp.float32(0.01))
                    sl = pl.ds(g * 64 + rr * 16, 16)
                    obuf[ko][r, sl] = wbuf[kc][r, sl] + pert
            return 0

        lax.fori_loop(0, _K, row_body, 0)

    @pl.when(eq)
    def _fast():
        fire_w(0, 0, 0)
        fire_e(0, 0, 0)

        def chunk_pair(ci, _):
            for kc in (0, 1):
                c = 2 * ci + kc

                @pl.when(c + 1 < _NCHUNK)
                def _():
                    fire_w(c + 1, 1 - kc, 0)

                wait_w(c, kc, 0)

                def p_pair(pj, _2):
                    for kp in (0, 1):
                        p = 2 * pj + kp
                        t = c * _P + p

                        @pl.when(t + 1 < _NSTEP)
                        def _():
                            fire_e(t + 1, 1 - kp, 0)

                        wait_e(kp)

                        @pl.when(t >= 2)
                        def _():
                            for bb in (0, 1):
                                pltpu.make_async_copy(
                                    obuf[kp],
                                    out_hbm.at[pl.ds(row_of(t - 2, bb), _K)],
                                    sem_o[kp]).wait()

                        compute(kc, kp, kp)
                        for bb in (0, 1):
                            pltpu.async_copy(
                                obuf[kp],
                                out_hbm.at[pl.ds(row_of(t, bb), _K)],
                                sem_o[kp])
                    return 0

                lax.fori_loop(0, _P // 2, p_pair, 0)
            return 0

        lax.fori_loop(0, _NCHUNK // 2, chunk_pair, 0)
        for t in (_NSTEP - 2, _NSTEP - 1):
            for bb in (0, 1):
                pltpu.make_async_copy(
                    obuf[t % 2], out_hbm.at[pl.ds(row_of(t, bb), _K)],
                    sem_o[t % 2]).wait()

    @pl.when(jnp.logical_not(eq))
    def _slow():
        def step_body(u, _):
            bb = u // _NSTEP
            t = u % _NSTEP
            c = t // _P
            p = t % _P

            @pl.when(p == 0)
            def _():
                fire_w(c, 0, bb)
                wait_w(c, 0, bb)

            fire_e(t, 0, bb)
            wait_e(0)
            compute(0, 0, 0)
            pltpu.sync_copy(obuf[0], out_hbm.at[pl.ds(row_of(t, bb), _K)])
            return 0

        lax.fori_loop(0, _B * _NSTEP, step_body, 0)


@functools.cache
def _sc_call():
    return pl.kernel(
        _sc_body,
        out_type=jax.ShapeDtypeStruct((_P * _B * _S, _D), jnp.float32),
        mesh=plsc.VectorSubcoreMesh(core_axis_name="c", subcore_axis_name="s",
                                    num_cores=_NC, num_subcores=_NS),
        compiler_params=pltpu.CompilerParams(needs_layout_passes=False),
        scratch_types=[
            pltpu.VMEM((_B, _S), jnp.int32),
            pltpu.VMEM((_B, _SB), jnp.int32),
            [pltpu.VMEM((_K,), jnp.int32)] * 2,
            [pltpu.VMEM((_K, _D), jnp.float32)] * 2,
            [pltpu.VMEM((_K, _DW), jnp.int32)] * 2,
            [pltpu.VMEM((_K, _D), jnp.float32)] * 2,
            [pltpu.SemaphoreType.DMA] * 2,
            [pltpu.SemaphoreType.DMA] * 2,
            [pltpu.SemaphoreType.DMA] * 2,
        ],
    )


def kernel(attention_mask, weight, past_key_values_length):
    del past_key_values_length
    mask = attention_mask.astype(jnp.int32)
    esign = _esign_on_device()
    out = _sc_call()(mask, weight.astype(jnp.float32), esign)
    return out.reshape(_P, _B, _S, _D)


# speedup vs baseline: 2.3661x; 1.4620x over previous
import functools

import jax
import jax.numpy as jnp
import numpy as np
from jax import lax
from jax.experimental import pallas as pl
from jax.experimental.pallas import tpu as pltpu
from jax.experimental.pallas import tpu_sc as plsc

_OFFSET = 2
_V = 2048 + _OFFSET
_D = 1024
_P = 8
_B = 2
_S = 2048

_NC = 2
_NS = 16
_NW = _NC * _NS
_SB = _S // _NW
_K = 16
_NCHUNK = _SB // _K
_NSTEP = _NCHUNK * _P

_DW = _P * (_D // 32)


def _packed_sign_table() -> np.ndarray:
    def draw():
        key = jax.random.key(42)
        return np.asarray(jax.random.randint(key, (_P, _V, _D), 0, 2))

    try:
        with jax.default_device(jax.devices("cpu")[0]):
            mu01 = draw()
    except Exception:
        try:
            mu01 = draw()
        except Exception:
            mu01 = np.zeros((_P, _V, _D), np.int64)
    neg = (1 - mu01).astype(np.uint32).reshape(_P, _V, 2, 32, 16)
    packed = (neg << np.arange(32, dtype=np.uint32)[None, None, None, :, None]
              ).sum(axis=3, dtype=np.uint32)
    packed = packed.transpose(1, 0, 2, 3)
    return np.ascontiguousarray(
        packed.reshape(_V, _DW).view(np.int32))


_ESIGN = _packed_sign_table()


def _sc_body(mask_hbm, w_hbm, emu_hbm, out_hbm,
             mask_v, pos_v, wbuf, ebuf, obuf, sem_w, sem_e, sem_o):
    cid = lax.axis_index("c")
    sid = lax.axis_index("s")
    wid = sid * _NC + cid
    s0 = wid * _SB
    c0 = wid * _NCHUNK

    pltpu.sync_copy(mask_hbm, mask_v)

    def scan_body(c, carry):
        cy0, cy1 = carry
        m0 = mask_v[0, pl.ds(c * 16, 16)]
        m1 = mask_v[1, pl.ds(c * 16, 16)]
        cs0 = plsc.cumsum(m0) + cy0
        cs1 = plsc.cumsum(m1) + cy1

        @pl.when(jnp.logical_and(c >= c0, c < c0 + _NCHUNK))
        def _():
            pos_v[0, pl.ds((c - c0) * 16, 16)] = cs0 * m0 + (_OFFSET - 1)
            pos_v[1, pl.ds((c - c0) * 16, 16)] = cs1 * m1 + (_OFFSET - 1)

        return cs0[15], cs1[15]

    lax.fori_loop(0, _S // 16, scan_body, (jnp.int32(0), jnp.int32(0)))

    eq = jnp.bool_(True)
    for cc in range(_NCHUNK):
        sl = pl.ds(cc * 16, 16)
        eq = jnp.logical_and(eq, jnp.all(pos_v[0, sl] == pos_v[1, sl]))

    def fire_w(c, kc, bb):
        pltpu.async_copy(w_hbm.at[pos_v.at[bb, pl.ds(c * _K, _K)]],
                         wbuf[kc], sem_w[kc])

    def wait_w(c, kc, bb):
        pltpu.make_async_copy(w_hbm.at[pos_v.at[bb, pl.ds(c * _K, _K)]],
                              wbuf[kc], sem_w[kc]).wait()

    def fire_e(c, ke, bb):
        pltpu.async_copy(emu_hbm.at[pos_v.at[bb, pl.ds(c * _K, _K)]],
                         ebuf[ke], sem_e[ke])

    def wait_e(c, ke, bb):
        pltpu.make_async_copy(emu_hbm.at[pos_v.at[bb, pl.ds(c * _K, _K)]],
                              ebuf[ke], sem_e[ke]).wait()

    def row_of(t, bb):
        c = t // _P
        p = t % _P
        return (p * _B + bb) * _S + s0 + c * _K

    def compute(p, kc, ko):
        def row_body(r, _3):
            wq0 = ebuf[kc][r, pl.ds(p * 32, 16)]
            wq1 = ebuf[kc][r, pl.ds(p * 32 + 16, 16)]
            for v in range(_D // 16):
                q, bbit = divmod(v, 32)
                shifted = jnp.left_shift(wq0 if q == 0 else wq1, 31 - bbit)
                pert = jnp.where(shifted < 0,
                                 jnp.float32(-0.01),
                                 jnp.float32(0.01))
                sl = pl.ds(v * 16, 16)
                obuf[ko][r, sl] = wbuf[kc][r, sl] + pert
            return 0

        lax.fori_loop(0, _K, row_body, 0)

    @pl.when(eq)
    def _fast():
        fire_w(0, 0, 0)
        fire_e(0, 0, 0)

        def chunk_pair(ci, _):
            for kc in (0, 1):
                c = 2 * ci + kc

                @pl.when(c + 1 < _NCHUNK)
                def _():
                    fire_w(c + 1, 1 - kc, 0)
                    fire_e(c + 1, 1 - kc, 0)

                wait_w(c, kc, 0)
                wait_e(c, kc, 0)

                def p_pair(pj, _2):
                    for kp in (0, 1):
                        p = 2 * pj + kp
                        t = c * _P + p

                        @pl.when(t >= 2)
                        def _():
                            for bb in (0, 1):
                                pltpu.make_async_copy(
                                    obuf[kp],
                                    out_hbm.at[pl.ds(row_of(t - 2, bb), _K)],
                                    sem_o[kp]).wait()

                        compute(p, kc, kp)
                        for bb in (0, 1):
                            pltpu.async_copy(
                                obuf[kp],
                                out_hbm.at[pl.ds(row_of(t, bb), _K)],
                                sem_o[kp])
                    return 0

                lax.fori_loop(0, _P // 2, p_pair, 0)
            return 0

        lax.fori_loop(0, _NCHUNK // 2, chunk_pair, 0)
        for t in (_NSTEP - 2, _NSTEP - 1):
            for bb in (0, 1):
                pltpu.make_async_copy(
                    obuf[t % 2], out_hbm.at[pl.ds(row_of(t, bb), _K)],
                    sem_o[t % 2]).wait()

    @pl.when(jnp.logical_not(eq))
    def _slow():
        def step_body(u, _):
            bb = u // _NSTEP
            t = u % _NSTEP
            c = t // _P
            p = t % _P

            @pl.when(p == 0)
            def _():
                fire_w(c, 0, bb)
                fire_e(c, 0, bb)
                wait_w(c, 0, bb)
                wait_e(c, 0, bb)

            compute(p, 0, 0)
            pltpu.sync_copy(obuf[0], out_hbm.at[pl.ds(row_of(t, bb), _K)])
            return 0

        lax.fori_loop(0, _B * _NSTEP, step_body, 0)


@functools.cache
def _sc_call():
    return pl.kernel(
        _sc_body,
        out_type=jax.ShapeDtypeStruct((_P * _B * _S, _D), jnp.float32),
        mesh=plsc.VectorSubcoreMesh(core_axis_name="c", subcore_axis_name="s",
                                    num_cores=_NC, num_subcores=_NS),
        compiler_params=pltpu.CompilerParams(needs_layout_passes=False),
        scratch_types=[
            pltpu.VMEM((_B, _S), jnp.int32),
            pltpu.VMEM((_B, _SB), jnp.int32),
            [pltpu.VMEM((_K, _D), jnp.float32)] * 2,
            [pltpu.VMEM((_K, _DW), jnp.int32)] * 2,
            [pltpu.VMEM((_K, _D), jnp.float32)] * 2,
            [pltpu.SemaphoreType.DMA] * 2,
            [pltpu.SemaphoreType.DMA] * 2,
            [pltpu.SemaphoreType.DMA] * 2,
        ],
    )


def kernel(attention_mask, weight, past_key_values_length):
    del past_key_values_length
    mask = attention_mask.astype(jnp.int32)
    esign = jnp.asarray(_ESIGN)
    out = _sc_call()(mask, weight.astype(jnp.float32), esign)
    return out.reshape(_P, _B, _S, _D)
